# Initial kernel scaffold; baseline (speedup 1.0000x reference)
import jax
import jax.numpy as jnp
from jax.experimental import pallas as pl


def kernel(embeddings, labels):
    raise NotImplementedError("write your pallas kernel here")


# speedup vs baseline: 2052.0577x; 2052.0577x over previous
import functools

import jax
import jax.numpy as jnp
from jax.experimental import pallas as pl

_MARGIN = 1.0
_NUM_CLASSES = 10


def _triplet_kernel(x_ref, lab_row_ref, lab_col_ref, out_ref):
    x = x_ref[:, :]
    lab_row = lab_row_ref[:, :]
    lab_col = lab_col_ref[:, :]
    B = x.shape[0]

    g = jax.lax.dot_general(
        x, x, (((1,), (1,)), ((), ())), preferred_element_type=jnp.float32
    )

    eye = (
        jax.lax.broadcasted_iota(jnp.int32, (B, B), 0)
        == jax.lax.broadcasted_iota(jnp.int32, (B, B), 1)
    )
    diag = jnp.where(eye, g, 0.0)
    n_col = jnp.sum(diag, axis=1, keepdims=True)
    n_row = jnp.sum(diag, axis=0, keepdims=True)
    d = n_col + n_row - 2.0 * g

    same = lab_col == lab_row

    dp = jnp.max(jnp.where(same, d, -jnp.inf), axis=1, keepdims=True)

    s_row = jnp.sum(d, axis=1, keepdims=True)
    best = jnp.full((B, 1), jnp.inf, dtype=jnp.float32)
    m = jnp.zeros((B, 1), dtype=jnp.int32)
    for c in range(_NUM_CLASSES):
        cs_c = jnp.sum(jnp.where(lab_row == c, d, 0.0), axis=1, keepdims=True)
        neg_c = s_row - cs_c
        better = neg_c < best
        best = jnp.where(better, neg_c, best)
        m = jnp.where(better, jnp.full((B, 1), c, jnp.int32), m)

    lt = lab_col < lab_row
    gt = lab_col > lab_row
    off_col = jnp.sum(gt.astype(jnp.int32), axis=1, keepdims=True)
    cnt_col = jnp.sum(same.astype(jnp.int32), axis=1, keepdims=True)
    p = m + jnp.where(m >= off_col, cnt_col, 0)

    idx_row = jax.lax.broadcasted_iota(jnp.int32, (B, B), 1)
    idx_col = jax.lax.broadcasted_iota(jnp.int32, (B, B), 0)
    before = jnp.logical_or(lt, jnp.logical_and(same, idx_col < idx_row))
    pos_row = jnp.sum(before.astype(jnp.int32), axis=0, keepdims=True)

    sel = pos_row == p
    dn = jnp.sum(jnp.where(sel, d, 0.0), axis=1, keepdims=True)

    hinge = jnp.maximum(dp - dn + _MARGIN, 0.0)
    out_ref[:, :] = jnp.sum(hinge, axis=0, keepdims=True)


@jax.jit
def kernel(embeddings, labels):
    B = embeddings.shape[0]
    lab_row = labels.reshape(1, B).astype(jnp.int32)
    lab_col = labels.reshape(B, 1).astype(jnp.int32)
    out = pl.pallas_call(
        _triplet_kernel,
        out_shape=jax.ShapeDtypeStruct((1, 1), jnp.float32),
    )(embeddings, lab_row, lab_col)
    return out.reshape(())
